# baseline (speedup 1.0000x reference)
import functools

import jax
import jax.numpy as jnp
from jax import lax
from jax.experimental import pallas as pl
from jax.experimental.pallas import tpu as pltpu
from jax.experimental.pallas import tpu_sc as plsc

BATCH = 16384
N_HID = 64
N_TAB = 4
LANES = 16
NUM_CORES = 2
NUM_SUBCORES = 16
NW = NUM_CORES * NUM_SUBCORES
BPW = BATCH // NW
GROUP = 16
GBUF = N_TAB * GROUP
SCHUNK = 128
NSCHUNK = BPW // SCHUNK
NGROUP = SCHUNK // GROUP

_mesh = plsc.VectorSubcoreMesh(core_axis_name="c", subcore_axis_name="s")


@functools.partial(
    pl.kernel,
    mesh=_mesh,
    out_type=jax.ShapeDtypeStruct((BATCH, N_HID), jnp.float32),
    scratch_types=[
        pltpu.VMEM((N_TAB, BPW), jnp.int32),
        pltpu.VMEM((GBUF, N_HID), jnp.float32),
        pltpu.VMEM((BPW, N_HID), jnp.float32),
        pltpu.SemaphoreType.DMA,
    ],
)
def _lookup_sum(tT, e0, e1, e2, e3, out, idx_v, rbuf, obuf, sem):
    wid = lax.axis_index("s") * NUM_CORES + lax.axis_index("c")
    base = wid * BPW
    tabs = (e0, e1, e2, e3)

    for k in range(N_TAB):
        pltpu.sync_copy(tT.at[k, pl.ds(base, BPW)], idx_v.at[k])

    def group_body(g, _):
        row0 = g * GROUP
        iv = [idx_v[k, pl.ds(row0, GROUP)] for k in range(N_TAB)]
        cps = []
        for k in range(N_TAB):
            for r2 in range(GROUP):
                cps.append(
                    pltpu.async_copy(tabs[k].at[iv[k][r2]],
                                     rbuf.at[k * GROUP + r2], sem))
        for cp in cps:
            cp.wait()
        for r2 in range(GROUP):
            for j in range(N_HID // LANES):
                o = j * LANES
                v = (rbuf[0 * GROUP + r2, pl.ds(o, LANES)]
                     + rbuf[1 * GROUP + r2, pl.ds(o, LANES)]
                     + rbuf[2 * GROUP + r2, pl.ds(o, LANES)]
                     + rbuf[3 * GROUP + r2, pl.ds(o, LANES)])
                obuf[row0 + r2, pl.ds(o, LANES)] = v
        return 0

    lax.fori_loop(0, BPW // GROUP, group_body, 0)
    pltpu.sync_copy(obuf, out.at[pl.ds(base, BPW)])


def kernel(t, emb0, emb1, emb2, emb3):
    tT = t.T.reshape(N_TAB, BATCH)
    return _lookup_sum(tT, emb0, emb1, emb2, emb3)


# speedup vs baseline: 1.1065x; 1.1065x over previous
import functools

import jax
import jax.numpy as jnp
from jax import lax
from jax.experimental import pallas as pl
from jax.experimental.pallas import tpu as pltpu
from jax.experimental.pallas import tpu_sc as plsc

BATCH = 16384
N_HID = 64
N_TAB = 4
LANES = 16
NUM_CORES = 2
NUM_SUBCORES = 16
NW = NUM_CORES * NUM_SUBCORES
BPW = BATCH // NW
GROUP = 16
GBUF = N_TAB * GROUP
SCHUNK = 128
NSCHUNK = BPW // SCHUNK
NGROUP = SCHUNK // GROUP

_mesh = plsc.VectorSubcoreMesh(core_axis_name="c", subcore_axis_name="s")


@functools.partial(
    pl.kernel,
    mesh=_mesh,
    out_type=jax.ShapeDtypeStruct((BATCH, N_HID), jnp.float32),
    scratch_types=[
        pltpu.VMEM((N_TAB, BPW), jnp.int32),
        pltpu.VMEM((GBUF, N_HID), jnp.float32),
        pltpu.VMEM((GBUF, N_HID), jnp.float32),
        pltpu.VMEM((BPW, N_HID), jnp.float32),
        pltpu.SemaphoreType.DMA,
        pltpu.SemaphoreType.DMA,
    ],
)
def _lookup_sum(tT, e0, e1, e2, e3, out, idx_v, rb0, rb1, obuf, sm0, sm1):
    wid = lax.axis_index("s") * NUM_CORES + lax.axis_index("c")
    base = wid * BPW
    tabs = (e0, e1, e2, e3)
    NGRP = BPW // GROUP

    for k in range(N_TAB):
        pltpu.sync_copy(tT.at[k, pl.ds(base, BPW)], idx_v.at[k])

    def enqueue(g, rbuf, sem):
        row0 = g * GROUP
        iv = [idx_v[k, pl.ds(row0, GROUP)] for k in range(N_TAB)]
        for k in range(N_TAB):
            for r2 in range(GROUP):
                pltpu.async_copy(tabs[k].at[iv[k][r2]],
                                 rbuf.at[k * GROUP + r2], sem)

    def drain_sum(g, rbuf, sem):
        pltpu.make_async_copy(e0.at[pl.ds(0, GBUF), :], rbuf, sem).wait()
        row0 = g * GROUP
        for r2 in range(GROUP):
            for j in range(N_HID // LANES):
                o = j * LANES
                v = (rbuf[0 * GROUP + r2, pl.ds(o, LANES)]
                     + rbuf[1 * GROUP + r2, pl.ds(o, LANES)]
                     + rbuf[2 * GROUP + r2, pl.ds(o, LANES)]
                     + rbuf[3 * GROUP + r2, pl.ds(o, LANES)])
                obuf[row0 + r2, pl.ds(o, LANES)] = v

    enqueue(0, rb0, sm0)

    def pair_body(gg, _):
        g0 = gg * 2
        enqueue(g0 + 1, rb1, sm1)
        drain_sum(g0, rb0, sm0)

        @pl.when(gg < NGRP // 2 - 1)
        def _():
            enqueue(g0 + 2, rb0, sm0)

        drain_sum(g0 + 1, rb1, sm1)
        return 0

    lax.fori_loop(0, NGRP // 2, pair_body, 0)
    pltpu.sync_copy(obuf, out.at[pl.ds(base, BPW)])


def kernel(t, emb0, emb1, emb2, emb3):
    tT = t.T.reshape(N_TAB, BATCH)
    return _lookup_sum(tT, emb0, emb1, emb2, emb3)
